# baseline (speedup 1.0000x reference)
import functools

import jax
import jax.numpy as jnp
from jax import lax
from jax.experimental import pallas as pl
from jax.experimental.pallas import tpu as pltpu
from jax.experimental.pallas import tpu_sc as plsc

PAD_IDX = 0
D = 64
B, L = 1024, 200
N = B * L
NH = N // 2
VOCAB2 = 500000
NC, NS = 2, 16
NW = NC * NS
CH = 128
ROWS_PER_W = NH // NW
NCH = ROWS_PER_W // CH
NCH_PAD = 32
NROW = N // 128
NROW_H = NH // 128
GB = 40
GRID_H = NROW_H // GB


def _sc_gather_pairs(table2, ids_w):
    mesh = plsc.VectorSubcoreMesh(core_axis_name="c", subcore_axis_name="s")

    @functools.partial(
        pl.kernel,
        mesh=mesh,
        out_type=jax.ShapeDtypeStruct((NH, 2 * D), jnp.float32),
        scratch_types=[
            pltpu.VMEM((NCH_PAD, CH), jnp.int32),
            pltpu.VMEM((CH, 2 * D), jnp.float32),
            pltpu.VMEM((CH, 2 * D), jnp.float32),
            pltpu.SemaphoreType.DMA,
            pltpu.SemaphoreType.DMA,
        ],
    )
    def k(ids_hbm, table_hbm, out_hbm, idx_v, buf0, buf1, sem0, sem1):
        wid = lax.axis_index("s") * NC + lax.axis_index("c")
        base = wid * ROWS_PER_W
        pltpu.sync_copy(ids_hbm.at[wid], idx_v)
        pltpu.async_copy(table_hbm.at[idx_v.at[0]], buf0, sem0)

        def body(j0, carry):
            j = 2 * j0
            pltpu.async_copy(table_hbm.at[idx_v.at[j + 1]], buf1, sem1)
            pltpu.make_async_copy(table_hbm.at[idx_v.at[j]], buf0, sem0).wait()
            pltpu.sync_copy(buf0, out_hbm.at[pl.ds(base + j * CH, CH)])

            @pl.when(2 * j0 + 2 < NCH)
            def _():
                pltpu.async_copy(table_hbm.at[idx_v.at[j + 2]], buf0, sem0)

            pltpu.make_async_copy(
                table_hbm.at[idx_v.at[j + 1]], buf1, sem1).wait()
            pltpu.sync_copy(buf1, out_hbm.at[pl.ds(base + (j + 1) * CH, CH)])
            return carry

        lax.fori_loop(0, NCH // 2, body, 0)
        if NCH % 2:
            j_last = NCH - 1
            pltpu.make_async_copy(
                table_hbm.at[idx_v.at[j_last]], buf0, sem0).wait()
            pltpu.sync_copy(buf0, out_hbm.at[pl.ds(base + j_last * CH, CH)])

    return k(ids_w, table2)


def _tc_body(p_ref, v1_ref, v2_ref, h_ref, s_ref, r_ref, o_ref):
    pv = p_ref[...]
    lane = lax.broadcasted_iota(jnp.int32, (GB, 128, 2 * D), 2)
    h3 = h_ref[...][..., None]
    hm = jnp.where(lane < D, 1.0 - h3, h3)
    a1 = (pv * (hm * v1_ref[...][..., None])).reshape(GB * 128, 2 * D)
    a2 = (pv * (hm * v2_ref[...][..., None])).reshape(GB * 128, 2 * D)
    raw = lax.dot_general(
        a1, s_ref[...], (((1,), (0,)), ((), ())),
        preferred_element_type=jnp.float32)
    tr = lax.dot_general(
        a2, r_ref[...], (((1,), (0,)), ((), ())),
        preferred_element_type=jnp.float32)
    o_ref[...] = (raw + tr).reshape(GB, 128, D)


_HALF_SPECS = [
    pl.BlockSpec((GB, 128, 2 * D), lambda i: (i, 0, 0)),
    pl.BlockSpec((GB, 128), lambda i: (i, 0)),
    pl.BlockSpec((GB, 128), lambda i: (i, 0)),
    pl.BlockSpec((GB, 128), lambda i: (i, 0)),
    pl.BlockSpec((2 * D, D), lambda i: (0, 0)),
    pl.BlockSpec((2 * D, D), lambda i: (0, 0)),
]


def _tc_transform_first(p3, v1, v2, h, s_mat, r2):
    return pl.pallas_call(
        _tc_body,
        grid=(GRID_H,),
        in_specs=_HALF_SPECS,
        out_specs=pl.BlockSpec((GB, 128, D), lambda i: (i, 0, 0)),
        out_shape=jax.ShapeDtypeStruct((NROW, 128, D), jnp.float32),
    )(p3, v1, v2, h, s_mat, r2)


def _tc_transform_second(base, p3, v1, v2, h, s_mat, r2):
    def body(base_ref, p_ref, v1_ref, v2_ref, h_ref, s_ref, r_ref, o_ref):
        del base_ref
        _tc_body(p_ref, v1_ref, v2_ref, h_ref, s_ref, r_ref, o_ref)

    return pl.pallas_call(
        body,
        grid=(GRID_H,),
        in_specs=[pl.BlockSpec((8, 128, D), lambda i: (0, 0, 0))] + _HALF_SPECS,
        out_specs=pl.BlockSpec((GB, 128, D), lambda i: (i + GRID_H, 0, 0)),
        out_shape=jax.ShapeDtypeStruct((NROW, 128, D), jnp.float32),
        input_output_aliases={0: 0},
    )(base, p3, v1, v2, h, s_mat, r2)


def kernel(input_ids, role_mask, table, R):
    ids_flat = input_ids.reshape(N).astype(jnp.int32)
    t2 = table.reshape(VOCAB2, 2 * D)

    half = (ids_flat & 1).astype(jnp.float32).reshape(NROW, 128)
    sel = (role_mask.reshape(NROW, 128) == 1).astype(jnp.float32)
    valid = (ids_flat.reshape(NROW, 128) != PAD_IDX).astype(jnp.float32)
    v1 = valid * (1.0 - sel)
    v2 = valid * sel
    eye = jnp.eye(D, dtype=jnp.float32)
    s_mat = jnp.concatenate([eye, eye], axis=0)
    r2 = jnp.concatenate([R.T, R.T], axis=0)

    pidx = (ids_flat >> 1).reshape(2, NW, NCH, CH)
    pads = ((0, 0), (0, NCH_PAD - NCH), (0, 0))
    pairs_a = _sc_gather_pairs(t2, jnp.pad(pidx[0], pads))
    pairs_b = _sc_gather_pairs(t2, jnp.pad(pidx[1], pads))

    out0 = _tc_transform_first(
        pairs_a.reshape(NROW_H, 128, 2 * D),
        v1[:NROW_H], v2[:NROW_H], half[:NROW_H], s_mat, r2)
    out = _tc_transform_second(
        out0,
        pairs_b.reshape(NROW_H, 128, 2 * D),
        v1[NROW_H:], v2[NROW_H:], half[NROW_H:], s_mat, r2)
    return out.reshape(B, L, D)


# speedup vs baseline: 1.0080x; 1.0080x over previous
import functools

import jax
import jax.numpy as jnp
from jax import lax
from jax.experimental import pallas as pl
from jax.experimental.pallas import tpu as pltpu
from jax.experimental.pallas import tpu_sc as plsc

PAD_IDX = 0
D = 64
B, L = 1024, 200
N = B * L
NH = N // 2
VOCAB2 = 500000
NC, NS = 2, 16
NW = NC * NS
CH = 128
ROWS_PER_W = NH // NW
NCH = ROWS_PER_W // CH
NCH_PAD = 32
NROW = N // 128
NROW_H = NH // 128
GB = 80
GRID_H = NROW_H // GB


def _sc_gather_pairs(table2, ids_w):
    mesh = plsc.VectorSubcoreMesh(core_axis_name="c", subcore_axis_name="s")

    @functools.partial(
        pl.kernel,
        mesh=mesh,
        out_type=jax.ShapeDtypeStruct((NH, 2 * D), jnp.float32),
        scratch_types=[
            pltpu.VMEM((NCH_PAD, CH), jnp.int32),
            pltpu.VMEM((CH, 2 * D), jnp.float32),
            pltpu.VMEM((CH, 2 * D), jnp.float32),
            pltpu.SemaphoreType.DMA,
            pltpu.SemaphoreType.DMA,
        ],
    )
    def k(ids_hbm, table_hbm, out_hbm, idx_v, buf0, buf1, sem0, sem1):
        wid = lax.axis_index("s") * NC + lax.axis_index("c")
        base = wid * ROWS_PER_W
        pltpu.sync_copy(ids_hbm.at[wid], idx_v)
        pltpu.async_copy(table_hbm.at[idx_v.at[0]], buf0, sem0)

        def body(j0, carry):
            j = 2 * j0
            pltpu.async_copy(table_hbm.at[idx_v.at[j + 1]], buf1, sem1)
            pltpu.make_async_copy(table_hbm.at[idx_v.at[j]], buf0, sem0).wait()
            pltpu.sync_copy(buf0, out_hbm.at[pl.ds(base + j * CH, CH)])

            @pl.when(2 * j0 + 2 < NCH)
            def _():
                pltpu.async_copy(table_hbm.at[idx_v.at[j + 2]], buf0, sem0)

            pltpu.make_async_copy(
                table_hbm.at[idx_v.at[j + 1]], buf1, sem1).wait()
            pltpu.sync_copy(buf1, out_hbm.at[pl.ds(base + (j + 1) * CH, CH)])
            return carry

        lax.fori_loop(0, NCH // 2, body, 0)
        if NCH % 2:
            j_last = NCH - 1
            pltpu.make_async_copy(
                table_hbm.at[idx_v.at[j_last]], buf0, sem0).wait()
            pltpu.sync_copy(buf0, out_hbm.at[pl.ds(base + j_last * CH, CH)])

    return k(ids_w, table2)


def _tc_body(p_ref, v1_ref, v2_ref, h_ref, s_ref, r_ref, o_ref):
    pv = p_ref[...]
    lane = lax.broadcasted_iota(jnp.int32, (GB, 128, 2 * D), 2)
    h3 = h_ref[...][..., None]
    hm = jnp.where(lane < D, 1.0 - h3, h3)
    a1 = (pv * (hm * v1_ref[...][..., None])).reshape(GB * 128, 2 * D)
    a2 = (pv * (hm * v2_ref[...][..., None])).reshape(GB * 128, 2 * D)
    raw = lax.dot_general(
        a1, s_ref[...], (((1,), (0,)), ((), ())),
        preferred_element_type=jnp.float32)
    tr = lax.dot_general(
        a2, r_ref[...], (((1,), (0,)), ((), ())),
        preferred_element_type=jnp.float32)
    o_ref[...] = (raw + tr).reshape(GB, 128, D)


_HALF_SPECS = [
    pl.BlockSpec((GB, 128, 2 * D), lambda i: (i, 0, 0)),
    pl.BlockSpec((GB, 128), lambda i: (i, 0)),
    pl.BlockSpec((GB, 128), lambda i: (i, 0)),
    pl.BlockSpec((GB, 128), lambda i: (i, 0)),
    pl.BlockSpec((2 * D, D), lambda i: (0, 0)),
    pl.BlockSpec((2 * D, D), lambda i: (0, 0)),
]


def _tc_transform_first(p3, v1, v2, h, s_mat, r2):
    return pl.pallas_call(
        _tc_body,
        grid=(GRID_H,),
        in_specs=_HALF_SPECS,
        out_specs=pl.BlockSpec((GB, 128, D), lambda i: (i, 0, 0)),
        out_shape=jax.ShapeDtypeStruct((NROW, 128, D), jnp.float32),
    )(p3, v1, v2, h, s_mat, r2)


def _tc_transform_second(base, p3, v1, v2, h, s_mat, r2):
    def body(base_ref, p_ref, v1_ref, v2_ref, h_ref, s_ref, r_ref, o_ref):
        del base_ref
        _tc_body(p_ref, v1_ref, v2_ref, h_ref, s_ref, r_ref, o_ref)

    return pl.pallas_call(
        body,
        grid=(GRID_H,),
        in_specs=[pl.BlockSpec((8, 128, D), lambda i: (0, 0, 0))] + _HALF_SPECS,
        out_specs=pl.BlockSpec((GB, 128, D), lambda i: (i + GRID_H, 0, 0)),
        out_shape=jax.ShapeDtypeStruct((NROW, 128, D), jnp.float32),
        input_output_aliases={0: 0},
    )(base, p3, v1, v2, h, s_mat, r2)


def kernel(input_ids, role_mask, table, R):
    ids_flat = input_ids.reshape(N).astype(jnp.int32)
    t2 = table.reshape(VOCAB2, 2 * D)

    half = (ids_flat & 1).astype(jnp.float32).reshape(NROW, 128)
    sel = (role_mask.reshape(NROW, 128) == 1).astype(jnp.float32)
    valid = (ids_flat.reshape(NROW, 128) != PAD_IDX).astype(jnp.float32)
    v1 = valid * (1.0 - sel)
    v2 = valid * sel
    eye = jnp.eye(D, dtype=jnp.float32)
    s_mat = jnp.concatenate([eye, eye], axis=0)
    r2 = jnp.concatenate([R.T, R.T], axis=0)

    pidx = (ids_flat >> 1).reshape(2, NW, NCH, CH)
    pads = ((0, 0), (0, NCH_PAD - NCH), (0, 0))
    pairs_a = _sc_gather_pairs(t2, jnp.pad(pidx[0], pads))
    pairs_b = _sc_gather_pairs(t2, jnp.pad(pidx[1], pads))

    out0 = _tc_transform_first(
        pairs_a.reshape(NROW_H, 128, 2 * D),
        v1[:NROW_H], v2[:NROW_H], half[:NROW_H], s_mat, r2)
    out = _tc_transform_second(
        out0,
        pairs_b.reshape(NROW_H, 128, 2 * D),
        v1[NROW_H:], v2[NROW_H:], half[NROW_H:], s_mat, r2)
    return out.reshape(B, L, D)
